# baseline (speedup 1.0000x reference)
import functools

import jax
import jax.numpy as jnp
from jax import lax
from jax.experimental import pallas as pl
from jax.experimental.pallas import tpu as pltpu
from jax.experimental.pallas import tpu_sc as plsc

_N_NODES = 10000
_D_FEAT = 256
_N_EDGES = 160000
_MARGIN = 1.0
_EPS = 1e-6

_NC = 2
_NS = 16
_NW = _NC * _NS
_E_PER_W = _N_EDGES // _NW
_CHUNK = 64
_NCHUNK = _E_PER_W // _CHUNK
_TAIL = _E_PER_W - _NCHUNK * _CHUNK
_LANES = 16
_DJ = _D_FEAT // _LANES
_PP = 8


def _sqdist_body(table, cidx_hbm, sqp_hbm,
                 cidx, rows, sqp_v, sem0, sem1):
    wid = lax.axis_index("s") * _NC + lax.axis_index("c")
    base = wid * _E_PER_W
    pltpu.sync_copy(cidx_hbm.at[pl.ds(base * 2, _E_PER_W * 2)], cidx)
    lane = lax.iota(jnp.int32, _LANES)
    lo_mask = lane < _PP

    def fire(c, slot, sem):
        pltpu.async_copy(table.at[cidx.at[pl.ds(c * 2 * _CHUNK, 2 * _CHUNK)]],
                         slot, sem)

    def wait(c, slot, sem):
        pltpu.make_async_copy(
            table.at[cidx.at[pl.ds(c * 2 * _CHUNK, 2 * _CHUNK)]],
            slot, sem).wait()

    def edge_acc(slot, e, dst_off):
        acc = jnp.zeros((_LANES,), jnp.float32)
        for j in range(_DJ):
            a = slot[e, pl.ds(j * _LANES, _LANES)]
            b = slot[dst_off + e, pl.ds(j * _LANES, _LANES)]
            d = a - b + _EPS
            acc = acc + d * d
        return acc

    def fold_store(slot, e_local, e_global, dst_off):
        acc0 = edge_acc(slot, e_local, dst_off)
        acc1 = edge_acc(slot, e_local + 1, dst_off)
        f0 = acc0 + lax.rev(acc0, (0,))
        f1 = acc1 + lax.rev(acc1, (0,))
        merged = jnp.where(lo_mask, f0, f1)
        sqp_v[pl.ds(e_global * _PP, _LANES)] = merged

    def compute(c, slot):
        cb = c * _CHUNK

        def pair_body(p, carry):
            e = p * 2
            fold_store(slot, e, cb + e, _CHUNK)
            return carry

        lax.fori_loop(0, _CHUNK // 2, pair_body, 0, unroll=4)

    fire(0, rows.at[0], sem0)

    def chunk_body(c, carry):
        is_even = (c % 2) == 0

        @pl.when(jnp.logical_and(is_even, c + 1 < _NCHUNK))
        def _():
            fire(c + 1, rows.at[1], sem1)

        @pl.when(jnp.logical_and(jnp.logical_not(is_even), c + 1 < _NCHUNK))
        def _():
            fire(c + 1, rows.at[0], sem0)

        @pl.when(is_even)
        def _():
            wait(c, rows.at[0], sem0)
            compute(c, rows.at[0])

        @pl.when(jnp.logical_not(is_even))
        def _():
            wait(c, rows.at[1], sem1)
            compute(c, rows.at[1])

        return carry

    lax.fori_loop(0, _NCHUNK, chunk_body, 0, unroll=False)

    tb = _NCHUNK * _CHUNK
    tail_rows = rows.at[0, pl.ds(0, 2 * _TAIL), :]
    pltpu.async_copy(
        table.at[cidx.at[pl.ds(tb * 2, 2 * _TAIL)]], tail_rows, sem0).wait()
    for p in range(_TAIL // 2):
        e = 2 * p
        fold_store(rows.at[0], e, tb + e, _TAIL)

    pltpu.sync_copy(sqp_v, sqp_hbm.at[pl.ds(base * _PP, _E_PER_W * _PP)])


@functools.cache
def _build_sqdist_sc():
    mesh = plsc.VectorSubcoreMesh(core_axis_name="c", subcore_axis_name="s")
    return pl.kernel(
        _sqdist_body,
        out_type=jax.ShapeDtypeStruct((_N_EDGES * _PP,), jnp.float32),
        mesh=mesh,
        scratch_types=[
            pltpu.VMEM((2 * _E_PER_W,), jnp.int32),
            pltpu.VMEM((2, 2 * _CHUNK, _D_FEAT), jnp.float32),
            pltpu.VMEM((_E_PER_W * _PP,), jnp.float32),
            pltpu.SemaphoreType.DMA,
            pltpu.SemaphoreType.DMA,
        ],
    )


def _counts_tc_body(out_ref, logterm_ref):
    x = out_ref[...]
    m = jnp.max(x, axis=1, keepdims=True)
    col = lax.broadcasted_iota(jnp.int32, (_N_NODES, _D_FEAT), 1)
    pred = jnp.min(jnp.where(x == m, col, _D_FEAT), axis=1, keepdims=True)
    counts = jnp.sum((pred == col).astype(jnp.float32), axis=0)
    log_term = jnp.log(jnp.float32(0.1)) + jnp.sum(jnp.log(counts))
    logterm_ref[...] = jnp.broadcast_to(log_term, (1, 1))


_counts_tc = pl.pallas_call(
    _counts_tc_body,
    out_shape=jax.ShapeDtypeStruct((1, 1), jnp.float32),
)


def _hinge_tc_body(sqp_ref, logterm_ref, loss_ref):
    p = sqp_ref[...]
    drow = lax.broadcasted_iota(jnp.int32, (_D_FEAT, _D_FEAT // _PP), 0)
    gcol = lax.broadcasted_iota(jnp.int32, (_D_FEAT, _D_FEAT // _PP), 1)
    s = ((drow // _PP) == gcol).astype(jnp.float32)
    sq = jax.lax.dot(p, s, precision=jax.lax.Precision.HIGHEST)
    dist = jnp.sqrt(sq)
    hinge = jnp.sum(jnp.maximum(jnp.float32(_MARGIN) - dist, 0.0))
    loss_ref[...] = hinge - logterm_ref[...]


_hinge_tc = pl.pallas_call(
    _hinge_tc_body,
    out_shape=jax.ShapeDtypeStruct((1, 1), jnp.float32),
)


def kernel(output, edgeindex):
    src = edgeindex[0]
    dst = edgeindex[1]
    nmain = _NCHUNK * _CHUNK
    src_w = src.reshape(_NW, _E_PER_W)
    dst_w = dst.reshape(_NW, _E_PER_W)
    main = jnp.concatenate(
        [src_w[:, :nmain].reshape(_NW, _NCHUNK, _CHUNK),
         dst_w[:, :nmain].reshape(_NW, _NCHUNK, _CHUNK)], axis=2)
    tail = jnp.concatenate([src_w[:, nmain:], dst_w[:, nmain:]], axis=1)
    cidx = jnp.concatenate(
        [main.reshape(_NW, -1), tail], axis=1).reshape(-1)
    sqp = _build_sqdist_sc()(output, cidx)
    log_term = _counts_tc(output)
    loss = _hinge_tc(sqp.reshape(_N_EDGES * _PP // _D_FEAT, _D_FEAT), log_term)
    return loss[0, 0]


# speedup vs baseline: 1.0237x; 1.0237x over previous
import functools

import jax
import jax.numpy as jnp
from jax import lax
from jax.experimental import pallas as pl
from jax.experimental.pallas import tpu as pltpu
from jax.experimental.pallas import tpu_sc as plsc

_N_NODES = 10000
_D_FEAT = 256
_N_EDGES = 160000
_MARGIN = 1.0
_EPS = 1e-6

_NC = 2
_NS = 16
_NW = _NC * _NS
_E_PER_W = _N_EDGES // _NW
_CHUNK = 72
_NCHUNK = _E_PER_W // _CHUNK
_TAIL = _E_PER_W - _NCHUNK * _CHUNK
_LANES = 16
_DJ = _D_FEAT // _LANES
_PP = 8


def _sqdist_body(table, src_hbm, dst_hbm, sqp_hbm,
                 sidx, didx, srows, drows, sqp_v,
                 sem_s0, sem_d0, sem_s1, sem_d1):
    wid = lax.axis_index("s") * _NC + lax.axis_index("c")
    base = wid * _E_PER_W
    pltpu.sync_copy(src_hbm.at[pl.ds(base, _E_PER_W)], sidx)
    pltpu.sync_copy(dst_hbm.at[pl.ds(base, _E_PER_W)], didx)
    lane = lax.iota(jnp.int32, _LANES)
    lo_mask = lane < _PP

    def fire(c, slot_srows, slot_drows, sem_s, sem_d):
        cb = c * _CHUNK
        pltpu.async_copy(table.at[sidx.at[pl.ds(cb, _CHUNK)]],
                         slot_srows, sem_s)
        pltpu.async_copy(table.at[didx.at[pl.ds(cb, _CHUNK)]],
                         slot_drows, sem_d)

    def wait(c, slot_srows, slot_drows, sem_s, sem_d):
        cb = c * _CHUNK
        pltpu.make_async_copy(table.at[sidx.at[pl.ds(cb, _CHUNK)]],
                              slot_srows, sem_s).wait()
        pltpu.make_async_copy(table.at[didx.at[pl.ds(cb, _CHUNK)]],
                              slot_drows, sem_d).wait()

    def edge_acc(slot_srows, slot_drows, e):
        acc = jnp.zeros((_LANES,), jnp.float32)
        for j in range(_DJ):
            a = slot_srows[e, pl.ds(j * _LANES, _LANES)]
            b = slot_drows[e, pl.ds(j * _LANES, _LANES)]
            d = a - b + _EPS
            acc = acc + d * d
        return acc

    def fold_store(slot_srows, slot_drows, e_local, e_global):
        acc0 = edge_acc(slot_srows, slot_drows, e_local)
        acc1 = edge_acc(slot_srows, slot_drows, e_local + 1)
        f0 = acc0 + lax.rev(acc0, (0,))
        f1 = acc1 + lax.rev(acc1, (0,))
        merged = jnp.where(lo_mask, f0, f1)
        sqp_v[pl.ds(e_global * _PP, _LANES)] = merged

    def compute(c, slot_srows, slot_drows):
        cb = c * _CHUNK

        def pair_body(p, carry):
            e = p * 2
            fold_store(slot_srows, slot_drows, e, cb + e)
            return carry

        lax.fori_loop(0, _CHUNK // 2, pair_body, 0, unroll=4)

    fire(0, srows.at[0], drows.at[0], sem_s0, sem_d0)

    def chunk_body(c, carry):
        is_even = (c % 2) == 0

        @pl.when(jnp.logical_and(is_even, c + 1 < _NCHUNK))
        def _():
            fire(c + 1, srows.at[1], drows.at[1], sem_s1, sem_d1)

        @pl.when(jnp.logical_and(jnp.logical_not(is_even), c + 1 < _NCHUNK))
        def _():
            fire(c + 1, srows.at[0], drows.at[0], sem_s0, sem_d0)

        @pl.when(is_even)
        def _():
            wait(c, srows.at[0], drows.at[0], sem_s0, sem_d0)
            compute(c, srows.at[0], drows.at[0])

        @pl.when(jnp.logical_not(is_even))
        def _():
            wait(c, srows.at[1], drows.at[1], sem_s1, sem_d1)
            compute(c, srows.at[1], drows.at[1])

        return carry

    lax.fori_loop(0, _NCHUNK, chunk_body, 0, unroll=False)

    tb = _NCHUNK * _CHUNK
    tail_s = srows.at[0, pl.ds(0, _TAIL), :]
    tail_d = drows.at[0, pl.ds(0, _TAIL), :]
    h1 = pltpu.async_copy(table.at[sidx.at[pl.ds(tb, _TAIL)]], tail_s, sem_s0)
    h2 = pltpu.async_copy(table.at[didx.at[pl.ds(tb, _TAIL)]], tail_d, sem_d0)
    h1.wait()
    h2.wait()
    for p in range(_TAIL // 2):
        e = 2 * p
        fold_store(srows.at[0], drows.at[0], e, tb + e)

    pltpu.sync_copy(sqp_v, sqp_hbm.at[pl.ds(base * _PP, _E_PER_W * _PP)])


@functools.cache
def _build_sqdist_sc():
    mesh = plsc.VectorSubcoreMesh(core_axis_name="c", subcore_axis_name="s")
    return pl.kernel(
        _sqdist_body,
        out_type=jax.ShapeDtypeStruct((_N_EDGES * _PP,), jnp.float32),
        mesh=mesh,
        scratch_types=[
            pltpu.VMEM((_E_PER_W,), jnp.int32),
            pltpu.VMEM((_E_PER_W,), jnp.int32),
            pltpu.VMEM((2, _CHUNK, _D_FEAT), jnp.float32),
            pltpu.VMEM((2, _CHUNK, _D_FEAT), jnp.float32),
            pltpu.VMEM((_E_PER_W * _PP,), jnp.float32),
            pltpu.SemaphoreType.DMA,
            pltpu.SemaphoreType.DMA,
            pltpu.SemaphoreType.DMA,
            pltpu.SemaphoreType.DMA,
        ],
    )


def _counts_tc_body(out_ref, logterm_ref):
    x = out_ref[...]
    m = jnp.max(x, axis=1, keepdims=True)
    col = lax.broadcasted_iota(jnp.int32, (_N_NODES, _D_FEAT), 1)
    pred = jnp.min(jnp.where(x == m, col, _D_FEAT), axis=1, keepdims=True)
    counts = jnp.sum((pred == col).astype(jnp.float32), axis=0)
    log_term = jnp.log(jnp.float32(0.1)) + jnp.sum(jnp.log(counts))
    logterm_ref[...] = jnp.broadcast_to(log_term, (1, 1))


_counts_tc = pl.pallas_call(
    _counts_tc_body,
    out_shape=jax.ShapeDtypeStruct((1, 1), jnp.float32),
)


def _hinge_tc_body(sqp_ref, logterm_ref, loss_ref):
    p = sqp_ref[...]
    drow = lax.broadcasted_iota(jnp.int32, (_D_FEAT, _D_FEAT // _PP), 0)
    gcol = lax.broadcasted_iota(jnp.int32, (_D_FEAT, _D_FEAT // _PP), 1)
    s = ((drow // _PP) == gcol).astype(jnp.float32)
    sq = jax.lax.dot(p, s, precision=jax.lax.Precision.HIGHEST)
    dist = jnp.sqrt(sq)
    hinge = jnp.sum(jnp.maximum(jnp.float32(_MARGIN) - dist, 0.0))
    loss_ref[...] = hinge - logterm_ref[...]


_hinge_tc = pl.pallas_call(
    _hinge_tc_body,
    out_shape=jax.ShapeDtypeStruct((1, 1), jnp.float32),
)


def kernel(output, edgeindex):
    src = edgeindex[0]
    dst = edgeindex[1]
    sqp = _build_sqdist_sc()(output, src, dst)
    log_term = _counts_tc(output)
    loss = _hinge_tc(sqp.reshape(_N_EDGES * _PP // _D_FEAT, _D_FEAT), log_term)
    return loss[0, 0]


# speedup vs baseline: 1.0261x; 1.0023x over previous
import functools

import jax
import jax.numpy as jnp
from jax import lax
from jax.experimental import pallas as pl
from jax.experimental.pallas import tpu as pltpu
from jax.experimental.pallas import tpu_sc as plsc

_N_NODES = 10000
_D_FEAT = 256
_N_EDGES = 160000
_MARGIN = 1.0
_EPS = 1e-6

_NC = 2
_NS = 16
_NW = _NC * _NS
_E_PER_W = _N_EDGES // _NW
_CHUNK = 64
_NCHUNK = _E_PER_W // _CHUNK
_TAIL = _E_PER_W - _NCHUNK * _CHUNK
_LANES = 16
_DJ = _D_FEAT // _LANES
_PP = 8


def _sqdist_body(table, src_hbm, dst_hbm, sqp_hbm,
                 sidx, didx, srows, drows, sqp_v,
                 sem_s0, sem_d0, sem_s1, sem_d1):
    wid = lax.axis_index("s") * _NC + lax.axis_index("c")
    base = wid * _E_PER_W
    pltpu.sync_copy(src_hbm.at[pl.ds(base, _E_PER_W)], sidx)
    pltpu.sync_copy(dst_hbm.at[pl.ds(base, _E_PER_W)], didx)
    lane = lax.iota(jnp.int32, _LANES)
    lo_mask = lane < _PP

    def fire(c, slot_srows, slot_drows, sem_s, sem_d):
        cb = c * _CHUNK
        pltpu.async_copy(table.at[sidx.at[pl.ds(cb, _CHUNK)]],
                         slot_srows, sem_s)
        pltpu.async_copy(table.at[didx.at[pl.ds(cb, _CHUNK)]],
                         slot_drows, sem_d)

    def wait(c, slot_srows, slot_drows, sem_s, sem_d):
        cb = c * _CHUNK
        pltpu.make_async_copy(table.at[sidx.at[pl.ds(cb, _CHUNK)]],
                              slot_srows, sem_s).wait()
        pltpu.make_async_copy(table.at[didx.at[pl.ds(cb, _CHUNK)]],
                              slot_drows, sem_d).wait()

    def edge_acc(slot_srows, slot_drows, e):
        acc = jnp.zeros((_LANES,), jnp.float32)
        for j in range(_DJ):
            a = slot_srows[e, pl.ds(j * _LANES, _LANES)]
            b = slot_drows[e, pl.ds(j * _LANES, _LANES)]
            d = a - b + _EPS
            acc = acc + d * d
        return acc

    def fold_store(slot_srows, slot_drows, e_local, e_global):
        acc0 = edge_acc(slot_srows, slot_drows, e_local)
        acc1 = edge_acc(slot_srows, slot_drows, e_local + 1)
        f0 = acc0 + lax.rev(acc0, (0,))
        f1 = acc1 + lax.rev(acc1, (0,))
        merged = jnp.where(lo_mask, f0, f1)
        sqp_v[pl.ds(e_global * _PP, _LANES)] = merged

    def compute(c, slot_srows, slot_drows):
        cb = c * _CHUNK

        def pair_body(p, carry):
            e = p * 2
            fold_store(slot_srows, slot_drows, e, cb + e)
            return carry

        lax.fori_loop(0, _CHUNK // 2, pair_body, 0, unroll=4)

    fire(0, srows.at[0], drows.at[0], sem_s0, sem_d0)

    def chunk_body(c, carry):
        is_even = (c % 2) == 0

        @pl.when(jnp.logical_and(is_even, c + 1 < _NCHUNK))
        def _():
            fire(c + 1, srows.at[1], drows.at[1], sem_s1, sem_d1)

        @pl.when(jnp.logical_and(jnp.logical_not(is_even), c + 1 < _NCHUNK))
        def _():
            fire(c + 1, srows.at[0], drows.at[0], sem_s0, sem_d0)

        @pl.when(is_even)
        def _():
            wait(c, srows.at[0], drows.at[0], sem_s0, sem_d0)
            compute(c, srows.at[0], drows.at[0])

        @pl.when(jnp.logical_not(is_even))
        def _():
            wait(c, srows.at[1], drows.at[1], sem_s1, sem_d1)
            compute(c, srows.at[1], drows.at[1])

        return carry

    lax.fori_loop(0, _NCHUNK, chunk_body, 0, unroll=False)

    tb = _NCHUNK * _CHUNK
    tail_s = srows.at[0, pl.ds(0, _TAIL), :]
    tail_d = drows.at[0, pl.ds(0, _TAIL), :]
    h1 = pltpu.async_copy(table.at[sidx.at[pl.ds(tb, _TAIL)]], tail_s, sem_s0)
    h2 = pltpu.async_copy(table.at[didx.at[pl.ds(tb, _TAIL)]], tail_d, sem_d0)
    h1.wait()
    h2.wait()
    for p in range(_TAIL // 2):
        e = 2 * p
        fold_store(srows.at[0], drows.at[0], e, tb + e)

    pltpu.sync_copy(sqp_v, sqp_hbm.at[pl.ds(base * _PP, _E_PER_W * _PP)])


@functools.cache
def _build_sqdist_sc():
    mesh = plsc.VectorSubcoreMesh(core_axis_name="c", subcore_axis_name="s")
    return pl.kernel(
        _sqdist_body,
        out_type=jax.ShapeDtypeStruct((_N_EDGES * _PP,), jnp.float32),
        mesh=mesh,
        scratch_types=[
            pltpu.VMEM((_E_PER_W,), jnp.int32),
            pltpu.VMEM((_E_PER_W,), jnp.int32),
            pltpu.VMEM((2, _CHUNK, _D_FEAT), jnp.float32),
            pltpu.VMEM((2, _CHUNK, _D_FEAT), jnp.float32),
            pltpu.VMEM((_E_PER_W * _PP,), jnp.float32),
            pltpu.SemaphoreType.DMA,
            pltpu.SemaphoreType.DMA,
            pltpu.SemaphoreType.DMA,
            pltpu.SemaphoreType.DMA,
        ],
    )


def _counts_tc_body(out_ref, logterm_ref):
    x = out_ref[...]
    m = jnp.max(x, axis=1, keepdims=True)
    col = lax.broadcasted_iota(jnp.int32, (_N_NODES, _D_FEAT), 1)
    pred = jnp.min(jnp.where(x == m, col, _D_FEAT), axis=1, keepdims=True)
    counts = jnp.sum((pred == col).astype(jnp.float32), axis=0)
    log_term = jnp.log(jnp.float32(0.1)) + jnp.sum(jnp.log(counts))
    logterm_ref[...] = jnp.broadcast_to(log_term, (1, 1))


_counts_tc = pl.pallas_call(
    _counts_tc_body,
    out_shape=jax.ShapeDtypeStruct((1, 1), jnp.float32),
)


def _hinge_tc_body(sqp_ref, logterm_ref, loss_ref):
    p = sqp_ref[...]
    drow = lax.broadcasted_iota(jnp.int32, (_D_FEAT, _D_FEAT // _PP), 0)
    gcol = lax.broadcasted_iota(jnp.int32, (_D_FEAT, _D_FEAT // _PP), 1)
    s = ((drow // _PP) == gcol).astype(jnp.float32)
    sq = jax.lax.dot(p, s, precision=jax.lax.Precision.HIGHEST)
    dist = jnp.sqrt(sq)
    hinge = jnp.sum(jnp.maximum(jnp.float32(_MARGIN) - dist, 0.0))
    loss_ref[...] = hinge - logterm_ref[...]


_hinge_tc = pl.pallas_call(
    _hinge_tc_body,
    out_shape=jax.ShapeDtypeStruct((1, 1), jnp.float32),
)


def kernel(output, edgeindex):
    src = edgeindex[0]
    dst = edgeindex[1]
    sqp = _build_sqdist_sc()(output, src, dst)
    log_term = _counts_tc(output)
    loss = _hinge_tc(sqp.reshape(_N_EDGES * _PP // _D_FEAT, _D_FEAT), log_term)
    return loss[0, 0]
